# baseline (speedup 1.0000x reference)
import dataclasses
import functools

import jax
import jax.numpy as jnp
from jax import lax
from jax.experimental import pallas as pl
from jax.experimental.pallas import tpu as pltpu
from jax.experimental.pallas import tpu_sc as plsc

N = 10000
D = 128
H = 128
C = 40
E = 320000

NC = 2
NS = 16
NW = NC * NS
CH = 64
NCHUNK = 160
HALF = NCHUNK // 2
EPW = NCHUNK * CH
EPAD = EPW * NW
NPAD = 10112
ZROWS = NPAD // NS

_mesh = plsc.VectorSubcoreMesh(core_axis_name="c", subcore_axis_name="s")


def _spmm_body(x_hbm, ei_hbm, z_hbm, out_hbm, sidx, didx, rows0, rows1,
               accum, sems):
    sg0, sg1, ss0, ss1 = sems.at[0], sems.at[1], sems.at[2], sems.at[3]
    cid = lax.axis_index("c")
    sid = lax.axis_index("s")
    wid = cid * NS + sid
    pltpu.sync_copy(z_hbm, accum.at[pl.ds(sid * ZROWS, ZROWS)])
    plsc.subcore_barrier()

    for half in range(2):
        pltpu.sync_copy(ei_hbm.at[0, wid, half], sidx)
        pltpu.sync_copy(ei_hbm.at[1, wid, half], didx)

        @pl.loop(0, HALF)
        def _(c):
            pltpu.sync_copy(x_hbm.at[sidx.at[c]], rows0)
            pltpu.sync_copy(rows0, accum.at[didx.at[c]], add=True)

    plsc.subcore_barrier()
    pltpu.sync_copy(accum.at[pl.ds(sid * ZROWS, ZROWS)],
                    out_hbm.at[cid, pl.ds(sid * ZROWS, ZROWS)])


_spmm = functools.partial(
    pl.kernel,
    out_type=jax.ShapeDtypeStruct((NC, NPAD, H), jnp.float32),
    mesh=_mesh,
    scratch_types=[
        pltpu.VMEM((HALF, CH), jnp.int32),
        pltpu.VMEM((HALF, CH), jnp.int32),
        pltpu.VMEM((CH, H), jnp.float32),
        pltpu.VMEM((CH, H), jnp.float32),
        pltpu.VMEM_SHARED((NPAD, H), jnp.float32),
        pltpu.SemaphoreType.DMA((4,)),
    ],
)(_spmm_body)


EG = EPW // 16


def _deg_body(ei_hbm, z_hbm, out_hbm, sidx, didx, histo, histi):
    cid = lax.axis_index("c")
    sid = lax.axis_index("s")
    wid = cid * NS + sid
    pltpu.sync_copy(ei_hbm.at[0, wid], sidx)
    pltpu.sync_copy(ei_hbm.at[1, wid], didx)
    pltpu.sync_copy(z_hbm, histo)
    pltpu.sync_copy(z_hbm, histi)
    ones16 = jnp.ones((16,), jnp.float32)

    @pl.loop(0, EG)
    def _(i):
        plsc.addupdate_scatter(histo, [sidx[pl.ds(i * 16, 16)]], ones16)
        plsc.addupdate_scatter(histi, [didx[pl.ds(i * 16, 16)]], ones16)

    pltpu.sync_copy(histo, out_hbm.at[0, cid, sid])
    pltpu.sync_copy(histi, out_hbm.at[1, cid, sid])


_deg_cp = pltpu.CompilerParams()
if "needs_layout_passes" in pltpu.CompilerParams.__dataclass_fields__:
    _deg_cp = dataclasses.replace(_deg_cp, needs_layout_passes=False)

_deg = functools.partial(
    pl.kernel,
    out_type=jax.ShapeDtypeStruct((2, NC, NS, NPAD), jnp.float32),
    mesh=_mesh,
    scratch_types=[
        pltpu.VMEM((EPW,), jnp.int32),
        pltpu.VMEM((EPW,), jnp.int32),
        pltpu.VMEM((NPAD,), jnp.float32),
        pltpu.VMEM((NPAD,), jnp.float32),
    ],
    compiler_params=_deg_cp,
)(_deg_body)


def _norm(deg):
    return jnp.where(deg > 0, lax.rsqrt(jnp.maximum(deg, 1.0)), 0.0)


def _prep_body(deg_ref, x_ref, ns_ref, nd_ref, x1_ref):
    d = deg_ref[...]
    d0 = d[0].reshape(NW, NPAD)
    d1 = d[1].reshape(NW, NPAD)
    onesw = jnp.ones((NW, 1), jnp.float32)
    dego = lax.dot_general(d0, onesw, (((0,), (0,)), ((), ())),
                           preferred_element_type=jnp.float32)[:N]
    degi = lax.dot_general(d1, onesw, (((0,), (0,)), ((), ())),
                           preferred_element_type=jnp.float32)[:N]
    ns = _norm(dego)
    nd = _norm(degi)
    ns_ref[...] = ns
    nd_ref[...] = nd
    x1_ref[:N] = x_ref[...] * ns
    x1_ref[pl.ds(N, NPAD - N), :] = jnp.zeros((NPAD - N, D), jnp.float32)


_prep = pl.pallas_call(
    _prep_body,
    out_shape=(
        jax.ShapeDtypeStruct((N, 1), jnp.float32),
        jax.ShapeDtypeStruct((N, 1), jnp.float32),
        jax.ShapeDtypeStruct((NPAD, D), jnp.float32),
    ),
)


def _leaky(h):
    return jnp.where(h >= 0, h, 0.01 * h)


def _dense_body(bn, p_ref, nd_ref, ns_ref, w_ref, b_ref, g_ref, be_ref, out_ref):
    agg = (p_ref[0][:N] + p_ref[1][:N]) * nd_ref[...]
    h = jnp.dot(agg, w_ref[...], preferred_element_type=jnp.float32) + b_ref[...]
    if bn:
        mu = jnp.mean(h, axis=0, keepdims=True)
        var = jnp.mean((h - mu) ** 2, axis=0, keepdims=True)
        h = (h - mu) * lax.rsqrt(var + 1e-5) * g_ref[...] + be_ref[...]
    h = _leaky(h)
    out_ref[:N] = h * ns_ref[...]
    out_ref[pl.ds(N, NPAD - N), :] = jnp.zeros((NPAD - N, H), jnp.float32)


_dense_bn = pl.pallas_call(
    functools.partial(_dense_body, True),
    out_shape=jax.ShapeDtypeStruct((NPAD, H), jnp.float32),
)
_dense_nobn = pl.pallas_call(
    functools.partial(_dense_body, False),
    out_shape=jax.ShapeDtypeStruct((NPAD, H), jnp.float32),
)


def _final_body(p_ref, nd_ref, w_ref, b_ref, wc_ref, bc_ref, out_ref):
    agg = (p_ref[0][:N] + p_ref[1][:N]) * nd_ref[...]
    h = jnp.dot(agg, w_ref[...], preferred_element_type=jnp.float32) + b_ref[...]
    h = _leaky(h)
    out_ref[...] = (
        jnp.dot(h, wc_ref[...], preferred_element_type=jnp.float32) + bc_ref[...]
    )


_final = pl.pallas_call(
    _final_body,
    out_shape=jax.ShapeDtypeStruct((N, C), jnp.float32),
)


def kernel(node_feat, edge_index, W1, b1, g1, be1, W2, b2, g2, be2,
           Wn1, bn1, Wn2, bn2, Wc, bc):
    ei32 = edge_index.astype(jnp.int32)
    ei_pad = jnp.concatenate(
        [ei32, jnp.full((2, EPAD - E), N, jnp.int32)], axis=1)
    ei = ei_pad.reshape(2, NW, 2, HALF, CH)
    ei_deg = ei_pad.reshape(2, NW, EPW)
    z128 = jnp.zeros((ZROWS, H), jnp.float32)
    zn = jnp.zeros((NPAD,), jnp.float32)

    deg = _deg(ei_deg, zn)
    ns, nd, x1 = _prep(deg, node_feat)

    p = _spmm(x1, ei, z128)
    h = _dense_bn(p, nd, ns, W1, b1.reshape(1, H), g1.reshape(1, H),
                  be1.reshape(1, H))
    p = _spmm(h, ei, z128)
    h = _dense_bn(p, nd, ns, W2, b2.reshape(1, H), g2.reshape(1, H),
                  be2.reshape(1, H))
    p = _spmm(h, ei, z128)
    h = _dense_nobn(p, nd, ns, Wn1, bn1.reshape(1, H), bn1.reshape(1, H),
                    bn1.reshape(1, H))
    p = _spmm(h, ei, z128)
    return _final(p, nd, Wn2, bn2.reshape(1, H), Wc, bc.reshape(1, C))


# speedup vs baseline: 3.4608x; 3.4608x over previous
import dataclasses
import functools

import jax
import jax.numpy as jnp
from jax import lax
from jax.experimental import pallas as pl
from jax.experimental.pallas import tpu as pltpu
from jax.experimental.pallas import tpu_sc as plsc

N = 10000
D = 128
H = 128
C = 40
E = 320000

NC = 2
NS = 16
NW = NC * NS
CH = 64
NCHUNK = 160
HALF = NCHUNK // 2
EPW = NCHUNK * CH
EPAD = EPW * NW
NPAD = 10112
ZROWS = NPAD // NS

_mesh = plsc.VectorSubcoreMesh(core_axis_name="c", subcore_axis_name="s")


def _spmm_body(x_hbm, ei_hbm, z_hbm, out_hbm, sidx, didx, rows0, rows1,
               accum, sems):
    sg0, sg1, ss0, ss1 = sems.at[0], sems.at[1], sems.at[2], sems.at[3]
    cid = lax.axis_index("c")
    sid = lax.axis_index("s")
    wid = cid * NS + sid
    pltpu.sync_copy(z_hbm, accum.at[pl.ds(sid * ZROWS, ZROWS)])
    plsc.subcore_barrier()

    for half in range(2):
        pltpu.sync_copy(ei_hbm.at[0, wid, half], sidx)
        pltpu.sync_copy(ei_hbm.at[1, wid, half], didx)
        pltpu.async_copy(x_hbm.at[sidx.at[0]], rows0, sg0)
        pltpu.async_copy(x_hbm.at[sidx.at[1]], rows1, sg1)

        @pl.loop(0, HALF // 2)
        def _(i):
            c0 = 2 * i
            c1 = 2 * i + 1
            pltpu.make_async_copy(x_hbm.at[sidx.at[c0]], rows0, sg0).wait()
            pltpu.sync_copy(rows0, accum.at[didx.at[c0]], add=True)

            @pl.when(c0 + 2 < HALF)
            def _():
                pltpu.async_copy(x_hbm.at[sidx.at[c0 + 2]], rows0, sg0)

            pltpu.make_async_copy(x_hbm.at[sidx.at[c1]], rows1, sg1).wait()
            pltpu.sync_copy(rows1, accum.at[didx.at[c1]], add=True)

            @pl.when(c1 + 2 < HALF)
            def _():
                pltpu.async_copy(x_hbm.at[sidx.at[c1 + 2]], rows1, sg1)

    plsc.subcore_barrier()
    pltpu.sync_copy(accum.at[pl.ds(sid * ZROWS, ZROWS)],
                    out_hbm.at[cid, pl.ds(sid * ZROWS, ZROWS)])


_spmm = functools.partial(
    pl.kernel,
    out_type=jax.ShapeDtypeStruct((NC, NPAD, H), jnp.float32),
    mesh=_mesh,
    scratch_types=[
        pltpu.VMEM((HALF, CH), jnp.int32),
        pltpu.VMEM((HALF, CH), jnp.int32),
        pltpu.VMEM((CH, H), jnp.float32),
        pltpu.VMEM((CH, H), jnp.float32),
        pltpu.VMEM_SHARED((NPAD, H), jnp.float32),
        pltpu.SemaphoreType.DMA((4,)),
    ],
)(_spmm_body)


EG = EPW // 16


def _deg_body(ei_hbm, z_hbm, out_hbm, sidx, didx, histo, histi):
    cid = lax.axis_index("c")
    sid = lax.axis_index("s")
    wid = cid * NS + sid
    pltpu.sync_copy(ei_hbm.at[0, wid], sidx)
    pltpu.sync_copy(ei_hbm.at[1, wid], didx)
    pltpu.sync_copy(z_hbm, histo)
    pltpu.sync_copy(z_hbm, histi)
    ones16 = jnp.ones((16,), jnp.float32)

    @pl.loop(0, EG)
    def _(i):
        plsc.addupdate_scatter(histo, [sidx[pl.ds(i * 16, 16)]], ones16)
        plsc.addupdate_scatter(histi, [didx[pl.ds(i * 16, 16)]], ones16)

    pltpu.sync_copy(histo, out_hbm.at[0, cid, sid])
    pltpu.sync_copy(histi, out_hbm.at[1, cid, sid])


_deg_cp = pltpu.CompilerParams()
if "needs_layout_passes" in pltpu.CompilerParams.__dataclass_fields__:
    _deg_cp = dataclasses.replace(_deg_cp, needs_layout_passes=False)

_deg = functools.partial(
    pl.kernel,
    out_type=jax.ShapeDtypeStruct((2, NC, NS, NPAD), jnp.float32),
    mesh=_mesh,
    scratch_types=[
        pltpu.VMEM((EPW,), jnp.int32),
        pltpu.VMEM((EPW,), jnp.int32),
        pltpu.VMEM((NPAD,), jnp.float32),
        pltpu.VMEM((NPAD,), jnp.float32),
    ],
    compiler_params=_deg_cp,
)(_deg_body)


def _norm(deg):
    return jnp.where(deg > 0, lax.rsqrt(jnp.maximum(deg, 1.0)), 0.0)


def _prep_body(deg_ref, x_ref, ns_ref, nd_ref, x1_ref):
    d = deg_ref[...]
    d0 = d[0].reshape(NW, NPAD)
    d1 = d[1].reshape(NW, NPAD)
    onesw = jnp.ones((NW, 1), jnp.float32)
    dego = lax.dot_general(d0, onesw, (((0,), (0,)), ((), ())),
                           preferred_element_type=jnp.float32)[:N]
    degi = lax.dot_general(d1, onesw, (((0,), (0,)), ((), ())),
                           preferred_element_type=jnp.float32)[:N]
    ns = _norm(dego)
    nd = _norm(degi)
    ns_ref[...] = ns
    nd_ref[...] = nd
    x1_ref[:N] = x_ref[...] * ns
    x1_ref[pl.ds(N, NPAD - N), :] = jnp.zeros((NPAD - N, D), jnp.float32)


_prep = pl.pallas_call(
    _prep_body,
    out_shape=(
        jax.ShapeDtypeStruct((N, 1), jnp.float32),
        jax.ShapeDtypeStruct((N, 1), jnp.float32),
        jax.ShapeDtypeStruct((NPAD, D), jnp.float32),
    ),
)


def _leaky(h):
    return jnp.where(h >= 0, h, 0.01 * h)


def _dense_body(bn, p_ref, nd_ref, ns_ref, w_ref, b_ref, g_ref, be_ref, out_ref):
    agg = (p_ref[0][:N] + p_ref[1][:N]) * nd_ref[...]
    h = jnp.dot(agg, w_ref[...], preferred_element_type=jnp.float32) + b_ref[...]
    if bn:
        mu = jnp.mean(h, axis=0, keepdims=True)
        var = jnp.mean((h - mu) ** 2, axis=0, keepdims=True)
        h = (h - mu) * lax.rsqrt(var + 1e-5) * g_ref[...] + be_ref[...]
    h = _leaky(h)
    out_ref[:N] = h * ns_ref[...]
    out_ref[pl.ds(N, NPAD - N), :] = jnp.zeros((NPAD - N, H), jnp.float32)


_dense_bn = pl.pallas_call(
    functools.partial(_dense_body, True),
    out_shape=jax.ShapeDtypeStruct((NPAD, H), jnp.float32),
)
_dense_nobn = pl.pallas_call(
    functools.partial(_dense_body, False),
    out_shape=jax.ShapeDtypeStruct((NPAD, H), jnp.float32),
)


def _final_body(p_ref, nd_ref, w_ref, b_ref, wc_ref, bc_ref, out_ref):
    agg = (p_ref[0][:N] + p_ref[1][:N]) * nd_ref[...]
    h = jnp.dot(agg, w_ref[...], preferred_element_type=jnp.float32) + b_ref[...]
    h = _leaky(h)
    out_ref[...] = (
        jnp.dot(h, wc_ref[...], preferred_element_type=jnp.float32) + bc_ref[...]
    )


_final = pl.pallas_call(
    _final_body,
    out_shape=jax.ShapeDtypeStruct((N, C), jnp.float32),
)


def kernel(node_feat, edge_index, W1, b1, g1, be1, W2, b2, g2, be2,
           Wn1, bn1, Wn2, bn2, Wc, bc):
    ei32 = edge_index.astype(jnp.int32)
    dummy = N + jnp.arange(EPAD - E, dtype=jnp.int32) % (NPAD - N)
    ei_pad = jnp.concatenate(
        [ei32, jnp.broadcast_to(dummy, (2, EPAD - E))], axis=1)
    ei = ei_pad.reshape(2, NW, 2, HALF, CH)
    ei_deg = ei_pad.reshape(2, NW, EPW)
    z128 = jnp.zeros((ZROWS, H), jnp.float32)
    zn = jnp.zeros((NPAD,), jnp.float32)

    deg = _deg(ei_deg, zn)
    ns, nd, x1 = _prep(deg, node_feat)

    p = _spmm(x1, ei, z128)
    h = _dense_bn(p, nd, ns, W1, b1.reshape(1, H), g1.reshape(1, H),
                  be1.reshape(1, H))
    p = _spmm(h, ei, z128)
    h = _dense_bn(p, nd, ns, W2, b2.reshape(1, H), g2.reshape(1, H),
                  be2.reshape(1, H))
    p = _spmm(h, ei, z128)
    h = _dense_nobn(p, nd, ns, Wn1, bn1.reshape(1, H), bn1.reshape(1, H),
                    bn1.reshape(1, H))
    p = _spmm(h, ei, z128)
    return _final(p, nd, Wn2, bn2.reshape(1, H), Wc, bc.reshape(1, C))


# speedup vs baseline: 3.5913x; 1.0377x over previous
import dataclasses
import functools

import jax
import jax.numpy as jnp
from jax import lax
from jax.experimental import pallas as pl
from jax.experimental.pallas import tpu as pltpu
from jax.experimental.pallas import tpu_sc as plsc

N = 10000
D = 128
H = 128
C = 40
E = 320000

NC = 2
NS = 16
NW = NC * NS
CH = 80
NCHUNK = 128
NST = 4
SPC = NCHUNK // NST
EPW = NCHUNK * CH
EPAD = EPW * NW
NPAD = 10112
ZROWS = NPAD // NS

_mesh = plsc.VectorSubcoreMesh(core_axis_name="c", subcore_axis_name="s")


def _spmm_body(x_hbm, ei_hbm, z_hbm, out_hbm, sidx, didx, rows0, rows1,
               accum, sems):
    sg0, sg1, ss0, ss1 = sems.at[0], sems.at[1], sems.at[2], sems.at[3]
    cid = lax.axis_index("c")
    sid = lax.axis_index("s")
    wid = cid * NS + sid
    pltpu.sync_copy(z_hbm, accum.at[pl.ds(sid * ZROWS, ZROWS)])
    plsc.subcore_barrier()

    for stage in range(NST):
        pltpu.sync_copy(ei_hbm.at[0, wid, stage], sidx)
        pltpu.sync_copy(ei_hbm.at[1, wid, stage], didx)
        pltpu.async_copy(x_hbm.at[sidx.at[0]], rows0, sg0)
        pltpu.async_copy(x_hbm.at[sidx.at[1]], rows1, sg1)

        @pl.loop(0, SPC // 2)
        def _(i):
            c0 = 2 * i
            c1 = 2 * i + 1
            pltpu.make_async_copy(x_hbm.at[sidx.at[c0]], rows0, sg0).wait()
            pltpu.sync_copy(rows0, accum.at[didx.at[c0]], add=True)

            @pl.when(c0 + 2 < SPC)
            def _():
                pltpu.async_copy(x_hbm.at[sidx.at[c0 + 2]], rows0, sg0)

            pltpu.make_async_copy(x_hbm.at[sidx.at[c1]], rows1, sg1).wait()
            pltpu.sync_copy(rows1, accum.at[didx.at[c1]], add=True)

            @pl.when(c1 + 2 < SPC)
            def _():
                pltpu.async_copy(x_hbm.at[sidx.at[c1 + 2]], rows1, sg1)

    plsc.subcore_barrier()
    pltpu.sync_copy(accum.at[pl.ds(sid * ZROWS, ZROWS)],
                    out_hbm.at[cid, pl.ds(sid * ZROWS, ZROWS)])


_spmm = functools.partial(
    pl.kernel,
    out_type=jax.ShapeDtypeStruct((NC, NPAD, H), jnp.float32),
    mesh=_mesh,
    scratch_types=[
        pltpu.VMEM((SPC, CH), jnp.int32),
        pltpu.VMEM((SPC, CH), jnp.int32),
        pltpu.VMEM((CH, H), jnp.float32),
        pltpu.VMEM((CH, H), jnp.float32),
        pltpu.VMEM_SHARED((NPAD, H), jnp.float32),
        pltpu.SemaphoreType.DMA((4,)),
    ],
)(_spmm_body)


EG = EPW // 16


def _deg_body(ei_hbm, z_hbm, out_hbm, sidx, didx, histo, histi):
    cid = lax.axis_index("c")
    sid = lax.axis_index("s")
    wid = cid * NS + sid
    pltpu.sync_copy(ei_hbm.at[0, wid], sidx)
    pltpu.sync_copy(ei_hbm.at[1, wid], didx)
    pltpu.sync_copy(z_hbm, histo)
    pltpu.sync_copy(z_hbm, histi)
    ones16 = jnp.ones((16,), jnp.float32)

    @pl.loop(0, EG)
    def _(i):
        plsc.addupdate_scatter(histo, [sidx[pl.ds(i * 16, 16)]], ones16)
        plsc.addupdate_scatter(histi, [didx[pl.ds(i * 16, 16)]], ones16)

    pltpu.sync_copy(histo, out_hbm.at[0, cid, sid])
    pltpu.sync_copy(histi, out_hbm.at[1, cid, sid])


_deg_cp = pltpu.CompilerParams()
if "needs_layout_passes" in pltpu.CompilerParams.__dataclass_fields__:
    _deg_cp = dataclasses.replace(_deg_cp, needs_layout_passes=False)

_deg = functools.partial(
    pl.kernel,
    out_type=jax.ShapeDtypeStruct((2, NC, NS, NPAD), jnp.float32),
    mesh=_mesh,
    scratch_types=[
        pltpu.VMEM((EPW,), jnp.int32),
        pltpu.VMEM((EPW,), jnp.int32),
        pltpu.VMEM((NPAD,), jnp.float32),
        pltpu.VMEM((NPAD,), jnp.float32),
    ],
    compiler_params=_deg_cp,
)(_deg_body)


def _norm(deg):
    return jnp.where(deg > 0, lax.rsqrt(jnp.maximum(deg, 1.0)), 0.0)


def _prep_body(deg_ref, x_ref, ns_ref, nd_ref, x1_ref):
    d = deg_ref[...]
    d0 = d[0].reshape(NW, NPAD)
    d1 = d[1].reshape(NW, NPAD)
    onesw = jnp.ones((NW, 1), jnp.float32)
    dego = lax.dot_general(d0, onesw, (((0,), (0,)), ((), ())),
                           preferred_element_type=jnp.float32)[:N]
    degi = lax.dot_general(d1, onesw, (((0,), (0,)), ((), ())),
                           preferred_element_type=jnp.float32)[:N]
    ns = _norm(dego)
    nd = _norm(degi)
    ns_ref[...] = ns
    nd_ref[...] = nd
    x1_ref[:N] = x_ref[...] * ns
    x1_ref[pl.ds(N, NPAD - N), :] = jnp.zeros((NPAD - N, D), jnp.float32)


_prep = pl.pallas_call(
    _prep_body,
    out_shape=(
        jax.ShapeDtypeStruct((N, 1), jnp.float32),
        jax.ShapeDtypeStruct((N, 1), jnp.float32),
        jax.ShapeDtypeStruct((NPAD, D), jnp.float32),
    ),
)


def _leaky(h):
    return jnp.where(h >= 0, h, 0.01 * h)


def _dense_body(bn, p_ref, nd_ref, ns_ref, w_ref, b_ref, g_ref, be_ref, out_ref):
    agg = (p_ref[0][:N] + p_ref[1][:N]) * nd_ref[...]
    h = jnp.dot(agg, w_ref[...], preferred_element_type=jnp.float32) + b_ref[...]
    if bn:
        mu = jnp.mean(h, axis=0, keepdims=True)
        var = jnp.mean((h - mu) ** 2, axis=0, keepdims=True)
        h = (h - mu) * lax.rsqrt(var + 1e-5) * g_ref[...] + be_ref[...]
    h = _leaky(h)
    out_ref[:N] = h * ns_ref[...]
    out_ref[pl.ds(N, NPAD - N), :] = jnp.zeros((NPAD - N, H), jnp.float32)


_dense_bn = pl.pallas_call(
    functools.partial(_dense_body, True),
    out_shape=jax.ShapeDtypeStruct((NPAD, H), jnp.float32),
)
_dense_nobn = pl.pallas_call(
    functools.partial(_dense_body, False),
    out_shape=jax.ShapeDtypeStruct((NPAD, H), jnp.float32),
)


def _final_body(p_ref, nd_ref, w_ref, b_ref, wc_ref, bc_ref, out_ref):
    agg = (p_ref[0][:N] + p_ref[1][:N]) * nd_ref[...]
    h = jnp.dot(agg, w_ref[...], preferred_element_type=jnp.float32) + b_ref[...]
    h = _leaky(h)
    out_ref[...] = (
        jnp.dot(h, wc_ref[...], preferred_element_type=jnp.float32) + bc_ref[...]
    )


_final = pl.pallas_call(
    _final_body,
    out_shape=jax.ShapeDtypeStruct((N, C), jnp.float32),
)


def kernel(node_feat, edge_index, W1, b1, g1, be1, W2, b2, g2, be2,
           Wn1, bn1, Wn2, bn2, Wc, bc):
    ei32 = edge_index.astype(jnp.int32)
    dummy = N + jnp.arange(EPAD - E, dtype=jnp.int32) % (NPAD - N)
    ei_pad = jnp.concatenate(
        [ei32, jnp.broadcast_to(dummy, (2, EPAD - E))], axis=1)
    ei = ei_pad.reshape(2, NW, NST, SPC, CH)
    ei_deg = ei_pad.reshape(2, NW, EPW)
    z128 = jnp.zeros((ZROWS, H), jnp.float32)
    zn = jnp.zeros((NPAD,), jnp.float32)

    deg = _deg(ei_deg, zn)
    ns, nd, x1 = _prep(deg, node_feat)

    p = _spmm(x1, ei, z128)
    h = _dense_bn(p, nd, ns, W1, b1.reshape(1, H), g1.reshape(1, H),
                  be1.reshape(1, H))
    p = _spmm(h, ei, z128)
    h = _dense_bn(p, nd, ns, W2, b2.reshape(1, H), g2.reshape(1, H),
                  be2.reshape(1, H))
    p = _spmm(h, ei, z128)
    h = _dense_nobn(p, nd, ns, Wn1, bn1.reshape(1, H), bn1.reshape(1, H),
                    bn1.reshape(1, H))
    p = _spmm(h, ei, z128)
    return _final(p, nd, Wn2, bn2.reshape(1, H), Wc, bc.reshape(1, C))
